# Initial kernel scaffold; baseline (speedup 1.0000x reference)
import jax
import jax.numpy as jnp
from jax.experimental import pallas as pl


def kernel(x, edge_index, W, att_src, att_dst, bias):
    raise NotImplementedError("write your pallas kernel here")


# speedup vs baseline: 1.0050x; 1.0050x over previous
import functools

import jax
import jax.numpy as jnp
from jax.experimental import pallas as pl
from jax.experimental.pallas import tpu as pltpu

N = 10000
E = 320000
IN_CH = 128
OUT_CH = 128
HEADS = 8
NEG_SLOPE = 0.2


def _proj_body(x_ref, w_ref, att_s_ref, att_d_ref, xp_ref, asrc_ref, adst_ref):
    xp = jnp.dot(x_ref[...], w_ref[...], preferred_element_type=jnp.float32)
    xp_ref[...] = xp
    bn = xp.shape[0]
    xph = xp.reshape(bn, HEADS, OUT_CH)
    asrc_ref[...] = jnp.sum(xph * att_s_ref[...][None], axis=-1)
    adst_ref[...] = jnp.sum(xph * att_d_ref[...][None], axis=-1)


def _project(x, W, att_src, att_dst):
    BN = 1000
    grid = (N // BN,)
    return pl.pallas_call(
        _proj_body,
        grid=grid,
        in_specs=[
            pl.BlockSpec((BN, IN_CH), lambda i: (i, 0)),
            pl.BlockSpec((IN_CH, HEADS * OUT_CH), lambda i: (0, 0)),
            pl.BlockSpec((HEADS, OUT_CH), lambda i: (0, 0)),
            pl.BlockSpec((HEADS, OUT_CH), lambda i: (0, 0)),
        ],
        out_specs=[
            pl.BlockSpec((BN, HEADS * OUT_CH), lambda i: (i, 0)),
            pl.BlockSpec((BN, HEADS), lambda i: (i, 0)),
            pl.BlockSpec((BN, HEADS), lambda i: (i, 0)),
        ],
        out_shape=[
            jax.ShapeDtypeStruct((N, HEADS * OUT_CH), jnp.float32),
            jax.ShapeDtypeStruct((N, HEADS), jnp.float32),
            jax.ShapeDtypeStruct((N, HEADS), jnp.float32),
        ],
    )(x, W, att_src, att_dst)


def kernel(x, edge_index, W, att_src, att_dst, bias):
    n = x.shape[0]
    xp_flat, a_src, a_dst = _project(x, W, att_src, att_dst)
    xp = xp_flat.reshape(n, HEADS, OUT_CH)
    loop = jnp.arange(n, dtype=edge_index.dtype)
    ei = jnp.concatenate([edge_index, jnp.stack([loop, loop], axis=0)], axis=1)
    src, dst = ei[0], ei[1]
    alpha = jax.nn.leaky_relu(a_src[src] + a_dst[dst], negative_slope=NEG_SLOPE)
    amax = jax.ops.segment_max(alpha, dst, num_segments=n)
    alpha = jnp.exp(alpha - amax[dst])
    denom = jax.ops.segment_sum(alpha, dst, num_segments=n)
    alpha = alpha / (denom[dst] + 1e-16)
    msg = xp[src] * alpha[:, :, None]
    out = jax.ops.segment_sum(msg, dst, num_segments=n)
    out = out.reshape(n, HEADS * OUT_CH) + bias
    return (out, ei, alpha)


# speedup vs baseline: 4.2333x; 4.2121x over previous
import functools

import jax
import jax.numpy as jnp
from jax import lax
from jax.experimental import pallas as pl
from jax.experimental.pallas import tpu as pltpu
from jax.experimental.pallas import tpu_sc as plsc

NN = 10000
EE = 320000
ET = EE + NN
IN_CH = 128
OUT_CH = 128
HEADS = 8
NEG_SLOPE = 0.2

BC = 32
NW = 32
EW = 10368
NCHUNK_W = EW // BC
E_PAD = EW * NW
EW_H = 20736
NCHUNK_C = EW_H // BC
N_D = 10112
RT = 632
N_PAD_ROWS = 112


def _proj_body(x_ref, w_ref, as_ref, ad_ref, xp_ref, ta_ref, td_ref):
    x_blk = x_ref[...]
    w = w_ref[...]
    att_s = as_ref[...]
    att_d = ad_ref[...]
    bn = x_blk.shape[0]
    xps = []
    acols_s = []
    acols_d = []
    for h in range(HEADS):
        wh = w[:, h * OUT_CH:(h + 1) * OUT_CH]
        xph = jnp.dot(x_blk, wh, preferred_element_type=jnp.float32)
        xps.append(xph)
        acols_s.append(jnp.sum(xph * att_s[h][None, :], axis=1, keepdims=True))
        acols_d.append(jnp.sum(xph * att_d[h][None, :], axis=1, keepdims=True))
    xp_ref[...] = jnp.stack(xps, axis=0)
    zpad = jnp.zeros((bn, 128 - HEADS), jnp.float32)
    ta_ref[...] = jnp.concatenate(acols_s + [zpad], axis=1)
    td_ref[...] = jnp.concatenate(acols_d + [zpad], axis=1)


def _project(x, W, att_src, att_dst):
    BN = 1000
    return pl.pallas_call(
        _proj_body,
        grid=(NN // BN,),
        in_specs=[
            pl.BlockSpec((BN, IN_CH), lambda i: (i, 0)),
            pl.BlockSpec((IN_CH, HEADS * OUT_CH), lambda i: (0, 0)),
            pl.BlockSpec((HEADS, OUT_CH), lambda i: (0, 0)),
            pl.BlockSpec((HEADS, OUT_CH), lambda i: (0, 0)),
        ],
        out_specs=[
            pl.BlockSpec((HEADS, BN, OUT_CH), lambda i: (0, i, 0)),
            pl.BlockSpec((BN, 128), lambda i: (i, 0)),
            pl.BlockSpec((BN, 128), lambda i: (i, 0)),
        ],
        out_shape=[
            jax.ShapeDtypeStruct((HEADS, NN, OUT_CH), jnp.float32),
            jax.ShapeDtypeStruct((NN, 128), jnp.float32),
            jax.ShapeDtypeStruct((NN, 128), jnp.float32),
        ],
    )(x, W, att_src, att_dst)


def _exp_body(src_hbm, dst_hbm, tabs_hbm, tabd_hbm, exp_hbm,
              si, di, g1, g2, ev, sem1, sem2):
    c = lax.axis_index("c")
    s = lax.axis_index("s")
    e0 = (s * 2 + c) * EW

    def _chunk(ci, carry):
        base = e0 + ci * BC
        pltpu.sync_copy(src_hbm.at[pl.ds(base, BC)], si)
        pltpu.sync_copy(dst_hbm.at[pl.ds(base, BC)], di)
        cp1 = pltpu.async_copy(tabs_hbm.at[si], g1, sem1)
        cp2 = pltpu.async_copy(tabd_hbm.at[di], g2, sem2)
        cp1.wait()
        cp2.wait()

        def _row(j, c2):
            v = g1[j, pl.ds(0, 16)] + g2[j, pl.ds(0, 16)]
            v = jnp.where(v > 0.0, v, v * NEG_SLOPE)
            ev[j] = jnp.exp(v)
            return c2
        lax.fori_loop(0, BC, _row, 0)
        pltpu.sync_copy(ev, exp_hbm.at[pl.ds(base, BC)])
        return carry
    lax.fori_loop(0, NCHUNK_W, _chunk, 0)


_exps = functools.partial(
    pl.kernel,
    out_type=jax.ShapeDtypeStruct((E_PAD, 16), jnp.float32),
    mesh=plsc.VectorSubcoreMesh(core_axis_name="c", subcore_axis_name="s"),
    scratch_types=[
        pltpu.VMEM((BC,), jnp.int32),
        pltpu.VMEM((BC,), jnp.int32),
        pltpu.VMEM((BC, 128), jnp.float32),
        pltpu.VMEM((BC, 128), jnp.float32),
        pltpu.VMEM((BC, 16), jnp.float32),
        pltpu.SemaphoreType.DMA,
        pltpu.SemaphoreType.DMA,
    ],
)(_exp_body)


def _message_body(src_hbm, dst_hbm, exp_hbm, xp_hbm, bias_hbm,
                  den_hbm, out_hbm,
                  si, di, abuf, rows, dband, bbuf, acc, sem):
    c = lax.axis_index("c")
    s = lax.axis_index("s")
    r0 = s * RT
    e0 = s * EW_H

    def _zero_acc():
        def _zrow(j, carry):
            for k in range(8):
                rows[j, pl.ds(k * 16, 16)] = jnp.zeros((16,), jnp.float32)
            return carry
        lax.fori_loop(0, BC, _zrow, 0)

        def _zslab(t, carry):
            pltpu.sync_copy(rows, acc.at[pl.ds(r0 + t * BC, BC)])
            return carry
        lax.fori_loop(0, RT // BC, _zslab, 0)
        pltpu.sync_copy(rows.at[pl.ds(0, RT % BC)],
                        acc.at[pl.ds(r0 + (RT // BC) * BC, RT % BC)])

    _zero_acc()
    plsc.subcore_barrier()

    def _den_chunk(ci, carry):
        base = e0 + ci * BC
        pltpu.sync_copy(dst_hbm.at[pl.ds(base, BC)], di)
        pltpu.sync_copy(exp_hbm.at[pl.ds(base, BC)], abuf)

        def _fill(j, c2):
            rows[j, pl.ds(0, 16)] = abuf[j]
            for k in range(1, 8):
                rows[j, pl.ds(k * 16, 16)] = jnp.zeros((16,), jnp.float32)
            return c2
        lax.fori_loop(0, BC, _fill, 0)
        pltpu.sync_copy(rows, acc.at[di], add=True)
        return carry
    lax.fori_loop(0, NCHUNK_C, _den_chunk, 0)
    plsc.subcore_barrier()

    def _den_flush(t, carry):
        sz = BC
        pltpu.sync_copy(acc.at[pl.ds(r0 + t * BC, sz)], rows.at[pl.ds(0, sz)])
        pltpu.sync_copy(rows.at[pl.ds(0, sz)],
                        den_hbm.at[c].at[pl.ds(r0 + t * BC, sz)])
        return carry
    lax.fori_loop(0, RT // BC, _den_flush, 0)
    szr = RT % BC
    pltpu.sync_copy(acc.at[pl.ds(r0 + (RT // BC) * BC, szr)],
                    rows.at[pl.ds(0, szr)])
    pltpu.sync_copy(rows.at[pl.ds(0, szr)],
                    den_hbm.at[c].at[pl.ds(r0 + (RT // BC) * BC, szr)])
    plsc.subcore_barrier()

    for hh in range(HEADS):
        @pl.when(c == hh // 4)
        def _head():
            _zero_acc()
            pltpu.sync_copy(bias_hbm.at[pl.ds(hh * OUT_CH, OUT_CH)], bbuf)
            plsc.subcore_barrier()

            def _msg_chunk(ci, carry):
                base = e0 + ci * BC
                pltpu.sync_copy(src_hbm.at[pl.ds(base, BC)], si)
                pltpu.sync_copy(dst_hbm.at[pl.ds(base, BC)], di)
                pltpu.sync_copy(exp_hbm.at[pl.ds(base, BC)], abuf)

                def _gidx(k, c2):
                    si[pl.ds(k * 16, 16)] = si[pl.ds(k * 16, 16)] + hh * NN
                    return c2
                lax.fori_loop(0, BC // 16, _gidx, 0)
                pltpu.async_copy(xp_hbm.at[si], rows, sem).wait()

                def _scale(j, c2):
                    w = abuf[j][hh]
                    for k in range(8):
                        rows[j, pl.ds(k * 16, 16)] = rows[j, pl.ds(k * 16, 16)] * w
                    return c2
                lax.fori_loop(0, BC, _scale, 0)
                pltpu.sync_copy(rows, acc.at[di], add=True)
                return carry
            lax.fori_loop(0, NCHUNK_C, _msg_chunk, 0)
            plsc.subcore_barrier()

            def _flush_rows(off, sz):
                pltpu.sync_copy(acc.at[pl.ds(r0 + off, sz)],
                                rows.at[pl.ds(0, sz)])
                pltpu.sync_copy(den_hbm.at[c].at[pl.ds(r0 + off, sz)],
                                dband.at[pl.ds(0, sz)])

                def _nrow(j, c2):
                    w = dband[j][hh] + 1e-16
                    for k in range(8):
                        rows[j, pl.ds(k * 16, 16)] = (
                            rows[j, pl.ds(k * 16, 16)] / w
                            + bbuf[pl.ds(k * 16, 16)])
                    return c2
                lax.fori_loop(0, sz, _nrow, 0)
                pltpu.sync_copy(rows.at[pl.ds(0, sz)],
                                out_hbm.at[hh].at[pl.ds(r0 + off, sz)])

            def _flush(t, carry):
                _flush_rows(t * BC, BC)
                return carry
            lax.fori_loop(0, RT // BC, _flush, 0)
            _flush_rows((RT // BC) * BC, RT % BC)
            plsc.subcore_barrier()


_messages = functools.partial(
    pl.kernel,
    out_type=[
        jax.ShapeDtypeStruct((2, N_D, 128), jnp.float32),
        jax.ShapeDtypeStruct((HEADS, N_D, OUT_CH), jnp.float32),
    ],
    mesh=plsc.VectorSubcoreMesh(core_axis_name="c", subcore_axis_name="s"),
    scratch_types=[
        pltpu.VMEM((BC,), jnp.int32),
        pltpu.VMEM((BC,), jnp.int32),
        pltpu.VMEM((BC, 16), jnp.float32),
        pltpu.VMEM((BC, OUT_CH), jnp.float32),
        pltpu.VMEM((BC, OUT_CH), jnp.float32),
        pltpu.VMEM((OUT_CH,), jnp.float32),
        pltpu.VMEM_SHARED((N_D, OUT_CH), jnp.float32),
        pltpu.SemaphoreType.DMA,
    ],
)(_message_body)


def _norm_body(dst_hbm, exp_hbm, den_hbm, alpha_hbm,
               di, ebuf, g1, sem1):
    c = lax.axis_index("c")
    s = lax.axis_index("s")
    e0 = (s * 2 + c) * EW

    def _chunk(ci, carry):
        base = e0 + ci * BC
        pltpu.sync_copy(dst_hbm.at[pl.ds(base, BC)], di)
        pltpu.sync_copy(exp_hbm.at[pl.ds(base, BC)], ebuf)
        pltpu.async_copy(den_hbm.at[di], g1, sem1).wait()

        def _row(j, c2):
            den = g1[j, pl.ds(0, 16)]
            ebuf[j] = ebuf[j] / (den + 1e-16)
            return c2
        lax.fori_loop(0, BC, _row, 0)
        pltpu.sync_copy(ebuf, alpha_hbm.at[pl.ds(base, BC)])
        return carry
    lax.fori_loop(0, NCHUNK_W, _chunk, 0)


_normalize = functools.partial(
    pl.kernel,
    out_type=jax.ShapeDtypeStruct((E_PAD, 16), jnp.float32),
    mesh=plsc.VectorSubcoreMesh(core_axis_name="c", subcore_axis_name="s"),
    scratch_types=[
        pltpu.VMEM((BC,), jnp.int32),
        pltpu.VMEM((BC, 16), jnp.float32),
        pltpu.VMEM((BC, 128), jnp.float32),
        pltpu.SemaphoreType.DMA,
    ],
)(_norm_body)


def kernel(x, edge_index, W, att_src, att_dst, bias):
    loop = jnp.arange(NN, dtype=edge_index.dtype)
    ei = jnp.concatenate([edge_index, jnp.stack([loop, loop], axis=0)], axis=1)

    npad = E_PAD - ET
    pad_i = jnp.arange(npad, dtype=jnp.int32)
    srcp = jnp.concatenate([ei[0], pad_i % NN])
    dstp = jnp.concatenate([ei[1], NN + pad_i % N_PAD_ROWS])

    xp_hm, ta, td = _project(x, W, att_src, att_dst)
    zrows = jnp.zeros((N_D - NN, 128), jnp.float32)
    ta = jnp.concatenate([ta, zrows], axis=0)
    td = jnp.concatenate([td, zrows], axis=0)
    xp_flat = xp_hm.reshape(HEADS * NN, OUT_CH)

    exp_rows = _exps(srcp, dstp, ta, td)
    den_hm, out_hm = _messages(srcp, dstp, exp_rows, xp_flat, bias)
    alpha_rows = _normalize(dstp, exp_rows, den_hm[0])

    alpha = alpha_rows[:ET, :HEADS]
    out = out_hm[:, :NN].transpose(1, 0, 2).reshape(NN, HEADS * OUT_CH)
    return (out, ei, alpha)


# speedup vs baseline: 5.2612x; 1.2428x over previous
import functools

import jax
import jax.numpy as jnp
from jax import lax
from jax.experimental import pallas as pl
from jax.experimental.pallas import tpu as pltpu
from jax.experimental.pallas import tpu_sc as plsc

NN = 10000
EE = 320000
ET = EE + NN
IN_CH = 128
OUT_CH = 128
HEADS = 8
NEG_SLOPE = 0.2

BC = 32
NW = 32
EW = 10368
NCHUNK_W = EW // BC
E_PAD = EW * NW
EW_H = 20736
NCHUNK_C = EW_H // BC
N_D = 10112
RT = 632
N_PAD_ROWS = 112


def _proj_body(x_ref, w_ref, as_ref, ad_ref, xp_ref, ta_ref, td_ref):
    x_blk = x_ref[...]
    w = w_ref[...]
    att_s = as_ref[...]
    att_d = ad_ref[...]
    bn = x_blk.shape[0]
    xps = []
    acols_s = []
    acols_d = []
    for h in range(HEADS):
        wh = w[:, h * OUT_CH:(h + 1) * OUT_CH]
        xph = jnp.dot(x_blk, wh, preferred_element_type=jnp.float32)
        xps.append(xph)
        acols_s.append(jnp.sum(xph * att_s[h][None, :], axis=1, keepdims=True))
        acols_d.append(jnp.sum(xph * att_d[h][None, :], axis=1, keepdims=True))
    xp_ref[...] = jnp.stack(xps, axis=0)
    zpad = jnp.zeros((bn, 128 - HEADS), jnp.float32)
    ta_ref[...] = jnp.concatenate(acols_s + [zpad], axis=1)
    td_ref[...] = jnp.concatenate(acols_d + [zpad], axis=1)


def _project(x, W, att_src, att_dst):
    BN = 1000
    return pl.pallas_call(
        _proj_body,
        grid=(NN // BN,),
        in_specs=[
            pl.BlockSpec((BN, IN_CH), lambda i: (i, 0)),
            pl.BlockSpec((IN_CH, HEADS * OUT_CH), lambda i: (0, 0)),
            pl.BlockSpec((HEADS, OUT_CH), lambda i: (0, 0)),
            pl.BlockSpec((HEADS, OUT_CH), lambda i: (0, 0)),
        ],
        out_specs=[
            pl.BlockSpec((HEADS, BN, OUT_CH), lambda i: (0, i, 0)),
            pl.BlockSpec((BN, 128), lambda i: (i, 0)),
            pl.BlockSpec((BN, 128), lambda i: (i, 0)),
        ],
        out_shape=[
            jax.ShapeDtypeStruct((HEADS, NN, OUT_CH), jnp.float32),
            jax.ShapeDtypeStruct((NN, 128), jnp.float32),
            jax.ShapeDtypeStruct((NN, 128), jnp.float32),
        ],
    )(x, W, att_src, att_dst)


def _exp_body(src_hbm, dst_hbm, tabs_hbm, tabd_hbm, exp_hbm,
              si, di, g1, g2, ev, sem1, sem2):
    c = lax.axis_index("c")
    s = lax.axis_index("s")
    e0 = (s * 2 + c) * EW

    def _chunk(ci, carry):
        base = e0 + ci * BC
        pltpu.sync_copy(src_hbm.at[pl.ds(base, BC)], si)
        pltpu.sync_copy(dst_hbm.at[pl.ds(base, BC)], di)
        cp1 = pltpu.async_copy(tabs_hbm.at[si], g1, sem1)
        cp2 = pltpu.async_copy(tabd_hbm.at[di], g2, sem2)
        cp1.wait()
        cp2.wait()

        def _row(j, c2):
            v = g1[j, pl.ds(0, 16)] + g2[j, pl.ds(0, 16)]
            v = jnp.where(v > 0.0, v, v * NEG_SLOPE)
            ev[j] = jnp.exp(v)
            return c2
        lax.fori_loop(0, BC, _row, 0)
        pltpu.sync_copy(ev, exp_hbm.at[pl.ds(base, BC)])
        return carry
    lax.fori_loop(0, NCHUNK_W, _chunk, 0)


_exps = functools.partial(
    pl.kernel,
    out_type=jax.ShapeDtypeStruct((E_PAD, 16), jnp.float32),
    mesh=plsc.VectorSubcoreMesh(core_axis_name="c", subcore_axis_name="s"),
    scratch_types=[
        pltpu.VMEM((BC,), jnp.int32),
        pltpu.VMEM((BC,), jnp.int32),
        pltpu.VMEM((BC, 128), jnp.float32),
        pltpu.VMEM((BC, 128), jnp.float32),
        pltpu.VMEM((BC, 16), jnp.float32),
        pltpu.SemaphoreType.DMA,
        pltpu.SemaphoreType.DMA,
    ],
)(_exp_body)


def _message_body(src_hbm, dst_hbm, exp_hbm, xp_hbm, bias_hbm,
                  den_hbm, out_hbm,
                  si0, di0, ab0, si1, di1, ab1, rows0, rows1,
                  dband, bbuf, acc, sem0, sem1):
    c = lax.axis_index("c")
    s = lax.axis_index("s")
    r0 = s * RT
    e0 = s * EW_H

    def _zero_acc():
        def _zrow(j, carry):
            for k in range(8):
                rows0[j, pl.ds(k * 16, 16)] = jnp.zeros((16,), jnp.float32)
            return carry
        lax.fori_loop(0, BC, _zrow, 0)

        def _zslab(t, carry):
            pltpu.sync_copy(rows0, acc.at[pl.ds(r0 + t * BC, BC)])
            return carry
        lax.fori_loop(0, RT // BC, _zslab, 0)
        pltpu.sync_copy(rows0.at[pl.ds(0, RT % BC)],
                        acc.at[pl.ds(r0 + (RT // BC) * BC, RT % BC)])

    _zero_acc()
    plsc.subcore_barrier()

    def _den_chunk(ci, carry):
        base = e0 + ci * BC
        pltpu.sync_copy(dst_hbm.at[pl.ds(base, BC)], di0)
        pltpu.sync_copy(exp_hbm.at[pl.ds(base, BC)], ab0)

        def _fill(j, c2):
            rows0[j, pl.ds(0, 16)] = ab0[j]
            return c2
        lax.fori_loop(0, BC, _fill, 0)
        pltpu.sync_copy(rows0, acc.at[di0], add=True)
        return carry
    lax.fori_loop(0, NCHUNK_C, _den_chunk, 0)
    plsc.subcore_barrier()

    def _den_flush(t, carry):
        sz = BC
        pltpu.sync_copy(acc.at[pl.ds(r0 + t * BC, sz)], rows0.at[pl.ds(0, sz)])
        pltpu.sync_copy(rows0.at[pl.ds(0, sz)],
                        den_hbm.at[c].at[pl.ds(r0 + t * BC, sz)])
        return carry
    lax.fori_loop(0, RT // BC, _den_flush, 0)
    szr = RT % BC
    pltpu.sync_copy(acc.at[pl.ds(r0 + (RT // BC) * BC, szr)],
                    rows0.at[pl.ds(0, szr)])
    pltpu.sync_copy(rows0.at[pl.ds(0, szr)],
                    den_hbm.at[c].at[pl.ds(r0 + (RT // BC) * BC, szr)])
    plsc.subcore_barrier()

    for hh in range(HEADS):
        @pl.when(c == hh // 4)
        def _head():
            _zero_acc()
            pltpu.sync_copy(bias_hbm.at[pl.ds(hh * OUT_CH, OUT_CH)], bbuf)
            plsc.subcore_barrier()

            def _prefetch(sib, dib, ab, rb, semb, ci):
                base = e0 + ci * BC
                pltpu.sync_copy(src_hbm.at[pl.ds(base, BC)], sib)
                pltpu.sync_copy(dst_hbm.at[pl.ds(base, BC)], dib)
                pltpu.sync_copy(exp_hbm.at[pl.ds(base, BC)], ab)

                def _gidx(k, c2):
                    sib[pl.ds(k * 16, 16)] = sib[pl.ds(k * 16, 16)] + hh * NN
                    return c2
                lax.fori_loop(0, BC // 16, _gidx, 0)
                pltpu.async_copy(xp_hbm.at[sib], rb, semb)

            def _stage(sib, dib, ab, rb, semb, pref):
                pltpu.make_async_copy(xp_hbm.at[sib], rb, semb).wait()

                def _scale(j, c2):
                    w = ab[j][hh]
                    for k in range(8):
                        rb[j, pl.ds(k * 16, 16)] = rb[j, pl.ds(k * 16, 16)] * w
                    return c2
                lax.fori_loop(0, BC, _scale, 0)
                pltpu.sync_copy(rb, acc.at[dib], add=True)
                _prefetch(sib, dib, ab, rb, semb, pref)

            _prefetch(si0, di0, ab0, rows0, sem0, 0)
            _prefetch(si1, di1, ab1, rows1, sem1, 1)

            def _pair(i2, carry):
                ci = i2 * 2
                _stage(si0, di0, ab0, rows0, sem0,
                       jnp.minimum(ci + 2, NCHUNK_C - 1))
                _stage(si1, di1, ab1, rows1, sem1,
                       jnp.minimum(ci + 3, NCHUNK_C - 1))
                return carry
            lax.fori_loop(0, NCHUNK_C // 2, _pair, 0)
            pltpu.make_async_copy(xp_hbm.at[si0], rows0, sem0).wait()
            pltpu.make_async_copy(xp_hbm.at[si1], rows1, sem1).wait()
            plsc.subcore_barrier()

            def _flush_rows(off, sz):
                pltpu.sync_copy(acc.at[pl.ds(r0 + off, sz)],
                                rows0.at[pl.ds(0, sz)])
                pltpu.sync_copy(den_hbm.at[c].at[pl.ds(r0 + off, sz)],
                                dband.at[pl.ds(0, sz)])

                def _nrow(j, c2):
                    w = dband[j][hh] + 1e-16
                    for k in range(8):
                        rows0[j, pl.ds(k * 16, 16)] = (
                            rows0[j, pl.ds(k * 16, 16)] / w
                            + bbuf[pl.ds(k * 16, 16)])
                    return c2
                lax.fori_loop(0, sz, _nrow, 0)
                pltpu.sync_copy(rows0.at[pl.ds(0, sz)],
                                out_hbm.at[hh].at[pl.ds(r0 + off, sz)])

            def _flush(t, carry):
                _flush_rows(t * BC, BC)
                return carry
            lax.fori_loop(0, RT // BC, _flush, 0)
            _flush_rows((RT // BC) * BC, RT % BC)
            plsc.subcore_barrier()


_messages = functools.partial(
    pl.kernel,
    out_type=[
        jax.ShapeDtypeStruct((2, N_D, 128), jnp.float32),
        jax.ShapeDtypeStruct((HEADS, N_D, OUT_CH), jnp.float32),
    ],
    mesh=plsc.VectorSubcoreMesh(core_axis_name="c", subcore_axis_name="s"),
    scratch_types=[
        pltpu.VMEM((BC,), jnp.int32),
        pltpu.VMEM((BC,), jnp.int32),
        pltpu.VMEM((BC, 16), jnp.float32),
        pltpu.VMEM((BC,), jnp.int32),
        pltpu.VMEM((BC,), jnp.int32),
        pltpu.VMEM((BC, 16), jnp.float32),
        pltpu.VMEM((BC, OUT_CH), jnp.float32),
        pltpu.VMEM((BC, OUT_CH), jnp.float32),
        pltpu.VMEM((BC, OUT_CH), jnp.float32),
        pltpu.VMEM((OUT_CH,), jnp.float32),
        pltpu.VMEM_SHARED((N_D, OUT_CH), jnp.float32),
        pltpu.SemaphoreType.DMA,
        pltpu.SemaphoreType.DMA,
    ],
)(_message_body)


def _norm_body(dst_hbm, exp_hbm, den_hbm, alpha_hbm,
               di, ebuf, g1, sem1):
    c = lax.axis_index("c")
    s = lax.axis_index("s")
    e0 = (s * 2 + c) * EW

    def _chunk(ci, carry):
        base = e0 + ci * BC
        pltpu.sync_copy(dst_hbm.at[pl.ds(base, BC)], di)
        pltpu.sync_copy(exp_hbm.at[pl.ds(base, BC)], ebuf)
        pltpu.async_copy(den_hbm.at[di], g1, sem1).wait()

        def _row(j, c2):
            den = g1[j, pl.ds(0, 16)]
            ebuf[j] = ebuf[j] / (den + 1e-16)
            return c2
        lax.fori_loop(0, BC, _row, 0)
        pltpu.sync_copy(ebuf, alpha_hbm.at[pl.ds(base, BC)])
        return carry
    lax.fori_loop(0, NCHUNK_W, _chunk, 0)


_normalize = functools.partial(
    pl.kernel,
    out_type=jax.ShapeDtypeStruct((E_PAD, 16), jnp.float32),
    mesh=plsc.VectorSubcoreMesh(core_axis_name="c", subcore_axis_name="s"),
    scratch_types=[
        pltpu.VMEM((BC,), jnp.int32),
        pltpu.VMEM((BC, 16), jnp.float32),
        pltpu.VMEM((BC, 128), jnp.float32),
        pltpu.SemaphoreType.DMA,
    ],
)(_norm_body)


def kernel(x, edge_index, W, att_src, att_dst, bias):
    loop = jnp.arange(NN, dtype=edge_index.dtype)
    ei = jnp.concatenate([edge_index, jnp.stack([loop, loop], axis=0)], axis=1)

    npad = E_PAD - ET
    pad_i = jnp.arange(npad, dtype=jnp.int32)
    srcp = jnp.concatenate([ei[0], pad_i % NN])
    dstp = jnp.concatenate([ei[1], NN + pad_i % N_PAD_ROWS])

    xp_hm, ta, td = _project(x, W, att_src, att_dst)
    zrows = jnp.zeros((N_D - NN, 128), jnp.float32)
    ta = jnp.concatenate([ta, zrows], axis=0)
    td = jnp.concatenate([td, zrows], axis=0)
    xp_flat = xp_hm.reshape(HEADS * NN, OUT_CH)

    exp_rows = _exps(srcp, dstp, ta, td)
    den_hm, out_hm = _messages(srcp, dstp, exp_rows, xp_flat, bias)
    alpha_rows = _normalize(dstp, exp_rows, den_hm[0])

    alpha = alpha_rows[:ET, :HEADS]
    out = out_hm[:, :NN].transpose(1, 0, 2).reshape(NN, HEADS * OUT_CH)
    return (out, ei, alpha)
